# baseline (speedup 1.0000x reference)
import functools

import jax
import jax.numpy as jnp
from jax import lax
from jax.experimental import pallas as pl
from jax.experimental.pallas import tpu as pltpu
from jax.experimental.pallas import tpu_sc as plsc

N = 50000
D = 256
B = 512
T = 3
DOUT = 256

NUM_WORKERS = 32
SEGS_PER_W = B // NUM_WORKERS
LANES = 16
DK = D // LANES
CHUNK = 128
MB = 4
NBUF = 2
NEG = -1e30

_GATHER_DNUMS = lax.GatherDimensionNumbers(
    offset_dims=(), collapsed_slice_dims=(0,), start_index_map=(0,))


def _shuffle(v, perm2d):
    return lax.gather(v, perm2d, _GATHER_DNUMS, (1,),
                      mode=lax.GatherScatterMode.PROMISE_IN_BOUNDS)


def _pool_body(x_hbm, starts_hbm, ends_hbm, q_hbm, r_hbm,
               starts_v, ends_v, qv, xbuf, outv, evm, dsem):
    wid = lax.axis_index("s") * 2 + lax.axis_index("c")
    base = wid * SEGS_PER_W
    pltpu.sync_copy(starts_hbm.at[pl.ds(base, SEGS_PER_W)],
                    starts_v.at[pl.ds(0, SEGS_PER_W)])
    pltpu.sync_copy(ends_hbm.at[pl.ds(base, SEGS_PER_W)],
                    ends_v.at[pl.ds(0, SEGS_PER_W)])
    pltpu.sync_copy(q_hbm.at[pl.ds(base, SEGS_PER_W)], qv)

    lane = lax.iota(jnp.int32, LANES)
    perms = [(lane ^ sh)[:, None] for sh in (8, 4, 2, 1)]
    zeros = jnp.zeros((LANES,), jnp.float32)
    m0 = jnp.full((LANES,), NEG, jnp.float32)

    t_start = starts_v[pl.ds(0, LANES)][0]
    t_end = ends_v[pl.ds(SEGS_PER_W - 1, LANES)][0]
    a0 = (t_start // 8) * 8
    n_flat = (t_end - a0 + CHUNK - 1) // CHUNK

    def dma_start(ci):
        a = a0 + ci * CHUNK
        a_dma = jnp.minimum(a, N - CHUNK)
        buf = ci % NBUF
        pltpu.make_async_copy(
            x_hbm.at[pl.ds(a_dma, CHUNK)], xbuf.at[buf], dsem.at[buf]
        ).start()

    for b in range(NBUF):
        @pl.when(b < n_flat)
        def _(b=b):
            dma_start(jnp.int32(b))

    def seg_body(j, w_in):
        s0 = starts_v[pl.ds(j, LANES)][0]
        e0 = ends_v[pl.ds(j, LANES)][0]
        qj = [qv[j, pl.ds(k * LANES, LANES)] for k in range(DK)]

        init = (w_in, m0, zeros) + tuple(zeros for _ in range(DK))
        g_lo = (s0 - a0) // CHUNK
        g_hi = jnp.where(e0 > s0, (e0 - a0 + CHUNK - 1) // CHUNK, g_lo)

        def chunk_body(g, carry):
            w, m_v, s_v = carry[0], carry[1], carry[2]
            rks = list(carry[3:])
            a = a0 + g * CHUNK
            a_dma = jnp.minimum(a, N - CHUNK)
            buf = g % NBUF

            @pl.when(g > w)
            def _():
                @pl.when(jnp.logical_and(g >= 1, g + NBUF - 1 < n_flat))
                def _():
                    dma_start(g + NBUF - 1)
                pltpu.make_async_copy(
                    x_hbm.at[pl.ds(a_dma, CHUNK)], xbuf.at[buf], dsem.at[buf]
                ).wait()

            w = jnp.maximum(w, g)
            lower = jnp.maximum(a, s0)

            def p1_body(t, rms):
                rm = list(rms)
                for ii in range(MB):
                    i = t * MB + ii
                    acc = [xbuf[buf, i, pl.ds(k * LANES, LANES)] * qj[k]
                           for k in range(DK)]
                    while len(acc) > 1:
                        acc = [acc[k] + acc[k + 1]
                               for k in range(0, len(acc), 2)]
                    acc = acc[0]
                    for perm in perms:
                        acc = acc + _shuffle(acc, perm)
                    idx = a_dma + i
                    valid = jnp.logical_and(idx >= lower, idx < e0)
                    gate = lax.convert_element_type(valid, jnp.float32)
                    gate_v = jnp.full((LANES,), gate)
                    e_eff = gate_v * acc + (gate_v - 1.0) * 3.0e38
                    evm[i, :] = e_eff
                    rm[ii] = jnp.maximum(rm[ii], e_eff)
                return tuple(rm)

            rm = lax.fori_loop(0, CHUNK // MB, p1_body, (m0,) * MB)
            mt = list(rm)
            while len(mt) > 1:
                mt = [jnp.maximum(mt[k], mt[k + 1])
                      for k in range(0, len(mt), 2)]
            m_new = jnp.maximum(m_v, mt[0])
            scale = jnp.exp(m_v - m_new)
            s_v = s_v * scale
            rks = [rk * scale for rk in rks]

            def p2_body(t, c2):
                s_c = c2[0]
                rr = list(c2[1:])
                ps = [jnp.exp(evm[t * MB + ii, :] - m_new)
                      for ii in range(MB)]
                pt = list(ps)
                while len(pt) > 1:
                    pt = [pt[k] + pt[k + 1] for k in range(0, len(pt), 2)]
                s_c = s_c + pt[0]
                for k in range(DK):
                    parts = [ps[ii] * xbuf[buf, t * MB + ii,
                                           pl.ds(k * LANES, LANES)]
                             for ii in range(MB)]
                    while len(parts) > 1:
                        parts = [parts[k2] + parts[k2 + 1]
                                 for k2 in range(0, len(parts), 2)]
                    rr[k] = rr[k] + parts[0]
                return (s_c,) + tuple(rr)

            fin2 = lax.fori_loop(0, CHUNK // MB, p2_body, (s_v,) + tuple(rks))
            return (w, m_new) + fin2

        fin = lax.fori_loop(g_lo, g_hi, chunk_body, init)
        inv = 1.0 / (fin[2] + 1e-16)
        for k in range(DK):
            outv[j, pl.ds(k * LANES, LANES)] = fin[3 + k] * inv
        return fin[0]

    w_fin = lax.fori_loop(0, SEGS_PER_W, seg_body, jnp.int32(-1))
    for b in range(NBUF):
        @pl.when(jnp.logical_and(b < n_flat, b > w_fin))
        def _(b=b):
            pltpu.make_async_copy(
                x_hbm.at[pl.ds(0, CHUNK)], xbuf.at[b], dsem.at[b]
            ).wait()

    pltpu.sync_copy(outv, r_hbm.at[pl.ds(base, SEGS_PER_W)])


_pool = functools.partial(
    pl.kernel,
    out_type=jax.ShapeDtypeStruct((B, D), jnp.float32),
    mesh=plsc.VectorSubcoreMesh(core_axis_name="c", subcore_axis_name="s"),
    scratch_types=[
        pltpu.VMEM((SEGS_PER_W + LANES,), jnp.int32),
        pltpu.VMEM((SEGS_PER_W + LANES,), jnp.int32),
        pltpu.VMEM((SEGS_PER_W, D), jnp.float32),
        pltpu.VMEM((NBUF, CHUNK, D), jnp.float32),
        pltpu.VMEM((SEGS_PER_W, D), jnp.float32),
        pltpu.VMEM((CHUNK, LANES), jnp.float32),
        pltpu.SemaphoreType.DMA((NBUF,)),
    ],
)(_pool_body)


def _lstm_body(q_ref, r_ref, h_ref, c_ref, wq_ref, wr_ref, whh_ref,
               bih_ref, bhh_ref, h_out, c_out):
    gates = (
        jnp.dot(q_ref[...], wq_ref[...], preferred_element_type=jnp.float32)
        + jnp.dot(r_ref[...], wr_ref[...], preferred_element_type=jnp.float32)
        + jnp.dot(h_ref[...], whh_ref[...], preferred_element_type=jnp.float32)
        + bih_ref[...] + bhh_ref[...]
    )
    i = jax.nn.sigmoid(gates[:, :D])
    f = jax.nn.sigmoid(gates[:, D:2 * D])
    g = jnp.tanh(gates[:, 2 * D:3 * D])
    o = jax.nn.sigmoid(gates[:, 3 * D:])
    c_new = f * c_ref[...] + i * g
    h_out[...] = o * jnp.tanh(c_new)
    c_out[...] = c_new


_lstm = pl.pallas_call(
    _lstm_body,
    out_shape=(
        jax.ShapeDtypeStruct((B, D), jnp.float32),
        jax.ShapeDtypeStruct((B, D), jnp.float32),
    ),
)


def _post_body(q_ref, r_ref, wp1_ref, wp2_ref, b_ref, o_ref):
    o_ref[...] = jnp.maximum(
        jnp.dot(q_ref[...], wp1_ref[...], preferred_element_type=jnp.float32)
        + jnp.dot(r_ref[...], wp2_ref[...], preferred_element_type=jnp.float32)
        + b_ref[...],
        0.0,
    )


_post = pl.pallas_call(
    _post_body,
    out_shape=jax.ShapeDtypeStruct((B, DOUT), jnp.float32),
)


def kernel(x, batch, W_ih, W_hh, b_ih, b_hh, W_post, b_post):
    x = x.astype(jnp.float32)
    b32 = batch.astype(jnp.int32)
    seg_ids = jnp.arange(B, dtype=jnp.int32)
    starts = jnp.searchsorted(b32, seg_ids, side="left").astype(jnp.int32)
    ends = jnp.searchsorted(b32, seg_ids, side="right").astype(jnp.int32)

    wih_t = W_ih.T
    wq = wih_t[:D]
    wr = wih_t[D:]
    whh_t = W_hh.T
    bih2 = b_ih.reshape(1, 4 * D)
    bhh2 = b_hh.reshape(1, 4 * D)
    wpost_t = W_post.T
    wp1 = wpost_t[:D]
    wp2 = wpost_t[D:]
    bpost2 = b_post.reshape(1, DOUT)

    q = jnp.zeros((B, D), jnp.float32)
    r = jnp.zeros((B, D), jnp.float32)
    h = jnp.zeros((B, D), jnp.float32)
    c = jnp.zeros((B, D), jnp.float32)
    for _ in range(T):
        h, c = _lstm(q, r, h, c, wq, wr, whh_t, bih2, bhh2)
        q = h
        r = _pool(x, starts, ends, q)
    return _post(q, r, wp1, wp2, bpost2)


# speedup vs baseline: 1.4003x; 1.4003x over previous
import functools

import jax
import jax.numpy as jnp
from jax import lax
from jax.experimental import pallas as pl
from jax.experimental.pallas import tpu as pltpu
from jax.experimental.pallas import tpu_sc as plsc

N = 50000
D = 256
B = 512
T = 3
DOUT = 256

NUM_WORKERS = 32
SEGS_PER_W = B // NUM_WORKERS
LANES = 16
DK = D // LANES
CHUNK = 32
MB = 4
NBUF = 8
NEG = -1e30

_GATHER_DNUMS = lax.GatherDimensionNumbers(
    offset_dims=(), collapsed_slice_dims=(0,), start_index_map=(0,))


def _shuffle(v, perm2d):
    return lax.gather(v, perm2d, _GATHER_DNUMS, (1,),
                      mode=lax.GatherScatterMode.PROMISE_IN_BOUNDS)


def _pool_body(x_hbm, starts_hbm, ends_hbm, q_hbm, r_hbm,
               starts_v, ends_v, qv, xbuf, outv, evm, dsem):
    wid = lax.axis_index("s") * 2 + lax.axis_index("c")
    base = wid * SEGS_PER_W
    pltpu.sync_copy(starts_hbm.at[pl.ds(base, SEGS_PER_W)],
                    starts_v.at[pl.ds(0, SEGS_PER_W)])
    pltpu.sync_copy(ends_hbm.at[pl.ds(base, SEGS_PER_W)],
                    ends_v.at[pl.ds(0, SEGS_PER_W)])
    pltpu.sync_copy(q_hbm.at[pl.ds(base, SEGS_PER_W)], qv)

    lane = lax.iota(jnp.int32, LANES)
    perms = [(lane ^ sh)[:, None] for sh in (8, 4, 2, 1)]
    zeros = jnp.zeros((LANES,), jnp.float32)
    m0 = jnp.full((LANES,), NEG, jnp.float32)

    t_start = starts_v[pl.ds(0, LANES)][0]
    t_end = ends_v[pl.ds(SEGS_PER_W - 1, LANES)][0]
    a0 = (t_start // 8) * 8
    n_flat = (t_end - a0 + CHUNK - 1) // CHUNK

    def dma_start(ci):
        a = a0 + ci * CHUNK
        a_dma = jnp.minimum(a, N - CHUNK)
        buf = ci % NBUF
        pltpu.make_async_copy(
            x_hbm.at[pl.ds(a_dma, CHUNK)], xbuf.at[buf], dsem.at[buf]
        ).start()

    for b in range(NBUF):
        @pl.when(b < n_flat)
        def _(b=b):
            dma_start(jnp.int32(b))

    def seg_body(j, w_in):
        s0 = starts_v[pl.ds(j, LANES)][0]
        e0 = ends_v[pl.ds(j, LANES)][0]
        qj = [qv[j, pl.ds(k * LANES, LANES)] for k in range(DK)]

        init = (w_in, m0, zeros) + tuple(zeros for _ in range(DK))
        g_lo = (s0 - a0) // CHUNK
        g_hi = jnp.where(e0 > s0, (e0 - a0 + CHUNK - 1) // CHUNK, g_lo)

        def chunk_body(g, carry):
            w, m_v, s_v = carry[0], carry[1], carry[2]
            rks = list(carry[3:])
            a = a0 + g * CHUNK
            a_dma = jnp.minimum(a, N - CHUNK)
            buf = g % NBUF

            @pl.when(g > w)
            def _():
                @pl.when(jnp.logical_and(g >= 1, g + NBUF - 1 < n_flat))
                def _():
                    dma_start(g + NBUF - 1)
                pltpu.make_async_copy(
                    x_hbm.at[pl.ds(a_dma, CHUNK)], xbuf.at[buf], dsem.at[buf]
                ).wait()

            w = jnp.maximum(w, g)
            lower = jnp.maximum(a, s0)

            def p1_body(t, rms):
                rm = list(rms)
                for ii in range(MB):
                    i = t * MB + ii
                    acc = [xbuf[buf, i, pl.ds(k * LANES, LANES)] * qj[k]
                           for k in range(DK)]
                    while len(acc) > 1:
                        acc = [acc[k] + acc[k + 1]
                               for k in range(0, len(acc), 2)]
                    acc = acc[0]
                    for perm in perms:
                        acc = acc + _shuffle(acc, perm)
                    idx = a_dma + i
                    valid = jnp.logical_and(idx >= lower, idx < e0)
                    gate = lax.convert_element_type(valid, jnp.float32)
                    gate_v = jnp.full((LANES,), gate)
                    e_eff = gate_v * acc + (gate_v - 1.0) * 3.0e38
                    evm[i, :] = e_eff
                    rm[ii] = jnp.maximum(rm[ii], e_eff)
                return tuple(rm)

            rm = lax.fori_loop(0, CHUNK // MB, p1_body, (m0,) * MB)
            mt = list(rm)
            while len(mt) > 1:
                mt = [jnp.maximum(mt[k], mt[k + 1])
                      for k in range(0, len(mt), 2)]
            m_new = jnp.maximum(m_v, mt[0])
            scale = jnp.exp(m_v - m_new)
            s_v = s_v * scale
            rks = [rk * scale for rk in rks]

            def p2_body(t, c2):
                s_c = c2[0]
                rr = list(c2[1:])
                ps = [jnp.exp(evm[t * MB + ii, :] - m_new)
                      for ii in range(MB)]
                pt = list(ps)
                while len(pt) > 1:
                    pt = [pt[k] + pt[k + 1] for k in range(0, len(pt), 2)]
                s_c = s_c + pt[0]
                for k in range(DK):
                    parts = [ps[ii] * xbuf[buf, t * MB + ii,
                                           pl.ds(k * LANES, LANES)]
                             for ii in range(MB)]
                    while len(parts) > 1:
                        parts = [parts[k2] + parts[k2 + 1]
                                 for k2 in range(0, len(parts), 2)]
                    rr[k] = rr[k] + parts[0]
                return (s_c,) + tuple(rr)

            fin2 = lax.fori_loop(0, CHUNK // MB, p2_body, (s_v,) + tuple(rks))
            return (w, m_new) + fin2

        fin = lax.fori_loop(g_lo, g_hi, chunk_body, init)
        inv = 1.0 / (fin[2] + 1e-16)
        for k in range(DK):
            outv[j, pl.ds(k * LANES, LANES)] = fin[3 + k] * inv
        return fin[0]

    w_fin = lax.fori_loop(0, SEGS_PER_W, seg_body, jnp.int32(-1))
    for b in range(NBUF):
        @pl.when(jnp.logical_and(b < n_flat, b > w_fin))
        def _(b=b):
            pltpu.make_async_copy(
                x_hbm.at[pl.ds(0, CHUNK)], xbuf.at[b], dsem.at[b]
            ).wait()

    pltpu.sync_copy(outv, r_hbm.at[pl.ds(base, SEGS_PER_W)])


_pool = functools.partial(
    pl.kernel,
    out_type=jax.ShapeDtypeStruct((B, D), jnp.float32),
    mesh=plsc.VectorSubcoreMesh(core_axis_name="c", subcore_axis_name="s"),
    scratch_types=[
        pltpu.VMEM((SEGS_PER_W + LANES,), jnp.int32),
        pltpu.VMEM((SEGS_PER_W + LANES,), jnp.int32),
        pltpu.VMEM((SEGS_PER_W, D), jnp.float32),
        pltpu.VMEM((NBUF, CHUNK, D), jnp.float32),
        pltpu.VMEM((SEGS_PER_W, D), jnp.float32),
        pltpu.VMEM((CHUNK, LANES), jnp.float32),
        pltpu.SemaphoreType.DMA((NBUF,)),
    ],
)(_pool_body)


def _lstm_body(q_ref, r_ref, h_ref, c_ref, wq_ref, wr_ref, whh_ref,
               bih_ref, bhh_ref, h_out, c_out):
    gates = (
        jnp.dot(q_ref[...], wq_ref[...], preferred_element_type=jnp.float32)
        + jnp.dot(r_ref[...], wr_ref[...], preferred_element_type=jnp.float32)
        + jnp.dot(h_ref[...], whh_ref[...], preferred_element_type=jnp.float32)
        + bih_ref[...] + bhh_ref[...]
    )
    i = jax.nn.sigmoid(gates[:, :D])
    f = jax.nn.sigmoid(gates[:, D:2 * D])
    g = jnp.tanh(gates[:, 2 * D:3 * D])
    o = jax.nn.sigmoid(gates[:, 3 * D:])
    c_new = f * c_ref[...] + i * g
    h_out[...] = o * jnp.tanh(c_new)
    c_out[...] = c_new


_lstm = pl.pallas_call(
    _lstm_body,
    out_shape=(
        jax.ShapeDtypeStruct((B, D), jnp.float32),
        jax.ShapeDtypeStruct((B, D), jnp.float32),
    ),
)


def _post_body(q_ref, r_ref, wp1_ref, wp2_ref, b_ref, o_ref):
    o_ref[...] = jnp.maximum(
        jnp.dot(q_ref[...], wp1_ref[...], preferred_element_type=jnp.float32)
        + jnp.dot(r_ref[...], wp2_ref[...], preferred_element_type=jnp.float32)
        + b_ref[...],
        0.0,
    )


_post = pl.pallas_call(
    _post_body,
    out_shape=jax.ShapeDtypeStruct((B, DOUT), jnp.float32),
)


def kernel(x, batch, W_ih, W_hh, b_ih, b_hh, W_post, b_post):
    x = x.astype(jnp.float32)
    b32 = batch.astype(jnp.int32)
    seg_ids = jnp.arange(B, dtype=jnp.int32)
    starts = jnp.searchsorted(b32, seg_ids, side="left").astype(jnp.int32)
    ends = jnp.searchsorted(b32, seg_ids, side="right").astype(jnp.int32)

    wih_t = W_ih.T
    wq = wih_t[:D]
    wr = wih_t[D:]
    whh_t = W_hh.T
    bih2 = b_ih.reshape(1, 4 * D)
    bhh2 = b_hh.reshape(1, 4 * D)
    wpost_t = W_post.T
    wp1 = wpost_t[:D]
    wp2 = wpost_t[D:]
    bpost2 = b_post.reshape(1, DOUT)

    q = jnp.zeros((B, D), jnp.float32)
    r = jnp.zeros((B, D), jnp.float32)
    h = jnp.zeros((B, D), jnp.float32)
    c = jnp.zeros((B, D), jnp.float32)
    for _ in range(T):
        h, c = _lstm(q, r, h, c, wq, wr, whh_t, bih2, bhh2)
        q = h
        r = _pool(x, starts, ends, q)
    return _post(q, r, wp1, wp2, bpost2)


# speedup vs baseline: 1.4802x; 1.0570x over previous
import functools

import jax
import jax.numpy as jnp
from jax import lax
from jax.experimental import pallas as pl
from jax.experimental.pallas import tpu as pltpu
from jax.experimental.pallas import tpu_sc as plsc

N = 50000
D = 256
B = 512
T = 3
DOUT = 256

NUM_WORKERS = 32
SEGS_PER_W = B // NUM_WORKERS
LANES = 16
DK = D // LANES
CHUNK = 16
MB = 4
NBUF = 12
NEG = -1e30

_GATHER_DNUMS = lax.GatherDimensionNumbers(
    offset_dims=(), collapsed_slice_dims=(0,), start_index_map=(0,))


def _shuffle(v, perm2d):
    return lax.gather(v, perm2d, _GATHER_DNUMS, (1,),
                      mode=lax.GatherScatterMode.PROMISE_IN_BOUNDS)


def _pool_body(x_hbm, starts_hbm, ends_hbm, q_hbm, r_hbm,
               starts_v, ends_v, qv, xbuf, outv, evm, dsem):
    wid = lax.axis_index("s") * 2 + lax.axis_index("c")
    base = wid * SEGS_PER_W
    pltpu.sync_copy(starts_hbm.at[pl.ds(base, SEGS_PER_W)],
                    starts_v.at[pl.ds(0, SEGS_PER_W)])
    pltpu.sync_copy(ends_hbm.at[pl.ds(base, SEGS_PER_W)],
                    ends_v.at[pl.ds(0, SEGS_PER_W)])
    pltpu.sync_copy(q_hbm.at[pl.ds(base, SEGS_PER_W)], qv)

    lane = lax.iota(jnp.int32, LANES)
    perms = [(lane ^ sh)[:, None] for sh in (8, 4, 2, 1)]
    zeros = jnp.zeros((LANES,), jnp.float32)
    m0 = jnp.full((LANES,), NEG, jnp.float32)

    t_start = starts_v[pl.ds(0, LANES)][0]
    t_end = ends_v[pl.ds(SEGS_PER_W - 1, LANES)][0]
    a0 = (t_start // 8) * 8
    n_flat = (t_end - a0 + CHUNK - 1) // CHUNK

    def dma_start(ci):
        a = a0 + ci * CHUNK
        a_dma = jnp.minimum(a, N - CHUNK)
        buf = ci % NBUF
        pltpu.make_async_copy(
            x_hbm.at[pl.ds(a_dma, CHUNK)], xbuf.at[buf], dsem.at[buf]
        ).start()

    for b in range(NBUF):
        @pl.when(b < n_flat)
        def _(b=b):
            dma_start(jnp.int32(b))

    def seg_body(j, w_in):
        s0 = starts_v[pl.ds(j, LANES)][0]
        e0 = ends_v[pl.ds(j, LANES)][0]
        qj = [qv[j, pl.ds(k * LANES, LANES)] for k in range(DK)]

        init = (w_in, m0, zeros) + tuple(zeros for _ in range(DK))
        g_lo = (s0 - a0) // CHUNK
        g_hi = jnp.where(e0 > s0, (e0 - a0 + CHUNK - 1) // CHUNK, g_lo)

        def chunk_body(g, carry):
            w, m_v, s_v = carry[0], carry[1], carry[2]
            rks = list(carry[3:])
            a = a0 + g * CHUNK
            a_dma = jnp.minimum(a, N - CHUNK)
            buf = g % NBUF

            @pl.when(g > w)
            def _():
                @pl.when(jnp.logical_and(g >= 1, g + NBUF - 1 < n_flat))
                def _():
                    dma_start(g + NBUF - 1)
                pltpu.make_async_copy(
                    x_hbm.at[pl.ds(a_dma, CHUNK)], xbuf.at[buf], dsem.at[buf]
                ).wait()

            w = jnp.maximum(w, g)
            lower = jnp.maximum(a, s0)

            def p1_body(t, rms):
                rm = list(rms)
                for ii in range(MB):
                    i = t * MB + ii
                    acc = [xbuf[buf, i, pl.ds(k * LANES, LANES)] * qj[k]
                           for k in range(DK)]
                    while len(acc) > 1:
                        acc = [acc[k] + acc[k + 1]
                               for k in range(0, len(acc), 2)]
                    acc = acc[0]
                    for perm in perms:
                        acc = acc + _shuffle(acc, perm)
                    idx = a_dma + i
                    valid = jnp.logical_and(idx >= lower, idx < e0)
                    gate = lax.convert_element_type(valid, jnp.float32)
                    gate_v = jnp.full((LANES,), gate)
                    e_eff = gate_v * acc + (gate_v - 1.0) * 3.0e38
                    evm[i, :] = e_eff
                    rm[ii] = jnp.maximum(rm[ii], e_eff)
                return tuple(rm)

            rm = lax.fori_loop(0, CHUNK // MB, p1_body, (m0,) * MB)
            mt = list(rm)
            while len(mt) > 1:
                mt = [jnp.maximum(mt[k], mt[k + 1])
                      for k in range(0, len(mt), 2)]
            m_new = jnp.maximum(m_v, mt[0])
            scale = jnp.exp(m_v - m_new)
            s_v = s_v * scale
            rks = [rk * scale for rk in rks]

            def p2_body(t, c2):
                s_c = c2[0]
                rr = list(c2[1:])
                ps = [jnp.exp(evm[t * MB + ii, :] - m_new)
                      for ii in range(MB)]
                pt = list(ps)
                while len(pt) > 1:
                    pt = [pt[k] + pt[k + 1] for k in range(0, len(pt), 2)]
                s_c = s_c + pt[0]
                for k in range(DK):
                    parts = [ps[ii] * xbuf[buf, t * MB + ii,
                                           pl.ds(k * LANES, LANES)]
                             for ii in range(MB)]
                    while len(parts) > 1:
                        parts = [parts[k2] + parts[k2 + 1]
                                 for k2 in range(0, len(parts), 2)]
                    rr[k] = rr[k] + parts[0]
                return (s_c,) + tuple(rr)

            fin2 = lax.fori_loop(0, CHUNK // MB, p2_body, (s_v,) + tuple(rks))
            return (w, m_new) + fin2

        fin = lax.fori_loop(g_lo, g_hi, chunk_body, init)
        inv = 1.0 / (fin[2] + 1e-16)
        for k in range(DK):
            outv[j, pl.ds(k * LANES, LANES)] = fin[3 + k] * inv
        return fin[0]

    w_fin = lax.fori_loop(0, SEGS_PER_W, seg_body, jnp.int32(-1))
    for b in range(NBUF):
        @pl.when(jnp.logical_and(b < n_flat, b > w_fin))
        def _(b=b):
            pltpu.make_async_copy(
                x_hbm.at[pl.ds(0, CHUNK)], xbuf.at[b], dsem.at[b]
            ).wait()

    pltpu.sync_copy(outv, r_hbm.at[pl.ds(base, SEGS_PER_W)])


_pool = functools.partial(
    pl.kernel,
    out_type=jax.ShapeDtypeStruct((B, D), jnp.float32),
    mesh=plsc.VectorSubcoreMesh(core_axis_name="c", subcore_axis_name="s"),
    scratch_types=[
        pltpu.VMEM((SEGS_PER_W + LANES,), jnp.int32),
        pltpu.VMEM((SEGS_PER_W + LANES,), jnp.int32),
        pltpu.VMEM((SEGS_PER_W, D), jnp.float32),
        pltpu.VMEM((NBUF, CHUNK, D), jnp.float32),
        pltpu.VMEM((SEGS_PER_W, D), jnp.float32),
        pltpu.VMEM((CHUNK, LANES), jnp.float32),
        pltpu.SemaphoreType.DMA((NBUF,)),
    ],
)(_pool_body)


def _lstm_body(q_ref, r_ref, h_ref, c_ref, wq_ref, wr_ref, whh_ref,
               bih_ref, bhh_ref, h_out, c_out):
    gates = (
        jnp.dot(q_ref[...], wq_ref[...], preferred_element_type=jnp.float32)
        + jnp.dot(r_ref[...], wr_ref[...], preferred_element_type=jnp.float32)
        + jnp.dot(h_ref[...], whh_ref[...], preferred_element_type=jnp.float32)
        + bih_ref[...] + bhh_ref[...]
    )
    i = jax.nn.sigmoid(gates[:, :D])
    f = jax.nn.sigmoid(gates[:, D:2 * D])
    g = jnp.tanh(gates[:, 2 * D:3 * D])
    o = jax.nn.sigmoid(gates[:, 3 * D:])
    c_new = f * c_ref[...] + i * g
    h_out[...] = o * jnp.tanh(c_new)
    c_out[...] = c_new


_lstm = pl.pallas_call(
    _lstm_body,
    out_shape=(
        jax.ShapeDtypeStruct((B, D), jnp.float32),
        jax.ShapeDtypeStruct((B, D), jnp.float32),
    ),
)


def _post_body(q_ref, r_ref, wp1_ref, wp2_ref, b_ref, o_ref):
    o_ref[...] = jnp.maximum(
        jnp.dot(q_ref[...], wp1_ref[...], preferred_element_type=jnp.float32)
        + jnp.dot(r_ref[...], wp2_ref[...], preferred_element_type=jnp.float32)
        + b_ref[...],
        0.0,
    )


_post = pl.pallas_call(
    _post_body,
    out_shape=jax.ShapeDtypeStruct((B, DOUT), jnp.float32),
)


def kernel(x, batch, W_ih, W_hh, b_ih, b_hh, W_post, b_post):
    x = x.astype(jnp.float32)
    b32 = batch.astype(jnp.int32)
    seg_ids = jnp.arange(B, dtype=jnp.int32)
    starts = jnp.searchsorted(b32, seg_ids, side="left").astype(jnp.int32)
    ends = jnp.searchsorted(b32, seg_ids, side="right").astype(jnp.int32)

    wih_t = W_ih.T
    wq = wih_t[:D]
    wr = wih_t[D:]
    whh_t = W_hh.T
    bih2 = b_ih.reshape(1, 4 * D)
    bhh2 = b_hh.reshape(1, 4 * D)
    wpost_t = W_post.T
    wp1 = wpost_t[:D]
    wp2 = wpost_t[D:]
    bpost2 = b_post.reshape(1, DOUT)

    q = jnp.zeros((B, D), jnp.float32)
    r = jnp.zeros((B, D), jnp.float32)
    h = jnp.zeros((B, D), jnp.float32)
    c = jnp.zeros((B, D), jnp.float32)
    for _ in range(T):
        h, c = _lstm(q, r, h, c, wq, wr, whh_t, bih2, bhh2)
        q = h
        r = _pool(x, starts, ends, q)
    return _post(q, r, wp1, wp2, bpost2)
